# baseline (speedup 1.0000x reference)
import functools

import jax
import jax.numpy as jnp
from jax import lax
from jax.experimental import pallas as pl
from jax.experimental.pallas import tpu as pltpu
from jax.experimental.pallas import tpu_sc as plsc

_N = 10000
_E = 160000
_F = 128
_NC = 2
_NS = 16
_W = _NC * _NS

_CH = 128
_EP = 163840
_EROWS = _EP // _CH
_TROWS = _EROWS // _NS
_DROWS = _EROWS // _W
_RPT = 9984 // _NS
_RTAIL = _N - _RPT * _NS

_mesh = plsc.VectorSubcoreMesh(core_axis_name="c", subcore_axis_name="s")


@functools.partial(
    pl.kernel,
    mesh=_mesh,
    out_type=[
        jax.ShapeDtypeStruct((_N, 16), jnp.float32),
        jax.ShapeDtypeStruct((_N, 16), jnp.float32),
    ],
    scratch_types=[
        pltpu.VMEM((_DROWS, _CH), jnp.int32),
        pltpu.VMEM((_CH, 16), jnp.float32),
        pltpu.VMEM((_CH, 16), jnp.float32),
        pltpu.VMEM_SHARED((_N + 1, 16), jnp.float32),
        pltpu.SemaphoreType.DMA,
    ],
)
def _deg_kernel(dstp_hbm, out0_hbm, out1_hbm, dst_v, ones_v, zeros_v,
                acc_sh, sem):
    c = lax.axis_index("c")
    s = lax.axis_index("s")
    w = s * _NC + c

    zero16f = jnp.zeros((16,), jnp.float32)
    ones16f = jnp.ones((16,), jnp.float32)

    def fbody(i, carry):
        ones_v[i, pl.ds(0, 16)] = ones16f
        zeros_v[i, pl.ds(0, 16)] = zero16f
        return carry

    lax.fori_loop(0, _CH, fbody, 0)

    r0 = s * _RPT
    for k in range(_RPT // _CH):
        pltpu.sync_copy(zeros_v, acc_sh.at[pl.ds(r0 + k * _CH, _CH)])
    rem = _RPT % _CH
    pltpu.sync_copy(zeros_v.at[pl.ds(0, rem)],
                    acc_sh.at[pl.ds(r0 + (_RPT // _CH) * _CH, rem)])

    @pl.when(s == 0)
    def _():
        pltpu.sync_copy(zeros_v.at[pl.ds(0, _RTAIL)],
                        acc_sh.at[pl.ds(_RPT * _NS, _RTAIL)])

    pltpu.sync_copy(dstp_hbm.at[pl.ds(w * _DROWS, _DROWS)], dst_v)

    plsc.subcore_barrier()

    def body(j, carry):
        pltpu.async_copy(ones_v, acc_sh.at[dst_v.at[j]], sem, add=True)
        return carry

    lax.fori_loop(0, _DROWS, body, 0)

    def dbody(j, carry):
        pltpu.make_async_copy(ones_v, acc_sh.at[dst_v.at[j]], sem).wait()
        return carry

    lax.fori_loop(0, _DROWS, dbody, 0)

    plsc.subcore_barrier()

    @pl.when(c == 0)
    def _():
        pltpu.sync_copy(acc_sh.at[pl.ds(r0, _RPT)],
                        out0_hbm.at[pl.ds(r0, _RPT)])

        @pl.when(s == 0)
        def _():
            pltpu.sync_copy(acc_sh.at[pl.ds(_RPT * _NS, _RTAIL)],
                            out0_hbm.at[pl.ds(_RPT * _NS, _RTAIL)])

    @pl.when(c == 1)
    def _():
        pltpu.sync_copy(acc_sh.at[pl.ds(r0, _RPT)],
                        out1_hbm.at[pl.ds(r0, _RPT)])

        @pl.when(s == 0)
        def _():
            pltpu.sync_copy(acc_sh.at[pl.ds(_RPT * _NS, _RTAIL)],
                            out1_hbm.at[pl.ds(_RPT * _NS, _RTAIL)])


@functools.partial(
    pl.kernel,
    mesh=_mesh,
    out_type=[
        jax.ShapeDtypeStruct((_N, _F), jnp.float32),
        jax.ShapeDtypeStruct((_N, _F), jnp.float32),
    ],
    scratch_types=[
        pltpu.VMEM((_TROWS // 2, _CH), jnp.int32),
        pltpu.VMEM((_TROWS // 2, _CH), jnp.int32),
        pltpu.VMEM((_CH, _F), jnp.float32),
        pltpu.VMEM((_CH, _F), jnp.float32),
        pltpu.VMEM_SHARED((_N + 1, _F), jnp.float32),
        pltpu.SemaphoreType.DMA,
        pltpu.SemaphoreType.DMA,
    ],
)
def _scatter_kernel(ulo_hbm, uhi_hbm, srcp_hbm, dstp_hbm, outlo_hbm,
                    outhi_hbm, src_v, dst_v, rows_v0, rows_v1, acc_sh,
                    sem0, sem1):
    c = lax.axis_index("c")
    s = lax.axis_index("s")

    def _fire(j, rows_ref, sem):
        @pl.when(c == 0)
        def _():
            pltpu.async_copy(ulo_hbm.at[src_v.at[j]], rows_ref, sem)

        @pl.when(c == 1)
        def _():
            pltpu.async_copy(uhi_hbm.at[src_v.at[j]], rows_ref, sem)

    def _drain(j, rows_ref, sem):
        @pl.when(c == 0)
        def _():
            pltpu.make_async_copy(ulo_hbm.at[src_v.at[j]], rows_ref,
                                  sem).wait()

        @pl.when(c == 1)
        def _():
            pltpu.make_async_copy(uhi_hbm.at[src_v.at[j]], rows_ref,
                                  sem).wait()

    zero16f = jnp.zeros((16,), jnp.float32)

    def zbody(i, carry):
        for j in range(_F // 16):
            rows_v0[i, pl.ds(j * 16, 16)] = zero16f
        return carry

    lax.fori_loop(0, _CH, zbody, 0)

    r0 = s * _RPT
    for k in range(_RPT // _CH):
        pltpu.sync_copy(rows_v0, acc_sh.at[pl.ds(r0 + k * _CH, _CH)])
    rem = _RPT % _CH
    pltpu.sync_copy(rows_v0.at[pl.ds(0, rem)],
                    acc_sh.at[pl.ds(r0 + (_RPT // _CH) * _CH, rem)])

    @pl.when(s == 0)
    def _():
        pltpu.sync_copy(rows_v0.at[pl.ds(0, _RTAIL)],
                        acc_sh.at[pl.ds(_RPT * _NS, _RTAIL)])

    plsc.subcore_barrier()

    _ph = _TROWS // 2
    for p in range(2):
        rb = s * _TROWS + p * _ph
        pltpu.sync_copy(srcp_hbm.at[pl.ds(rb, _ph)], src_v)
        pltpu.sync_copy(dstp_hbm.at[pl.ds(rb, _ph)], dst_v)

        _fire(0, rows_v0, sem0)

        def body(k, carry):
            _fire(2 * k + 1, rows_v1, sem1)

            _drain(2 * k, rows_v0, sem0)
            pltpu.sync_copy(rows_v0, acc_sh.at[dst_v.at[2 * k]], add=True)

            @pl.when(k < _ph // 2 - 1)
            def _():
                _fire(2 * k + 2, rows_v0, sem0)

            _drain(2 * k + 1, rows_v1, sem1)
            pltpu.sync_copy(rows_v1, acc_sh.at[dst_v.at[2 * k + 1]],
                            add=True)
            return carry

        lax.fori_loop(0, _ph // 2, body, 0)

    plsc.subcore_barrier()

    @pl.when(c == 0)
    def _():
        pltpu.sync_copy(acc_sh.at[pl.ds(r0, _RPT)],
                        outlo_hbm.at[pl.ds(r0, _RPT)])

        @pl.when(s == 0)
        def _():
            pltpu.sync_copy(acc_sh.at[pl.ds(_RPT * _NS, _RTAIL)],
                            outlo_hbm.at[pl.ds(_RPT * _NS, _RTAIL)])

    @pl.when(c == 1)
    def _():
        pltpu.sync_copy(acc_sh.at[pl.ds(r0, _RPT)],
                        outhi_hbm.at[pl.ds(r0, _RPT)])

        @pl.when(s == 0)
        def _():
            pltpu.sync_copy(acc_sh.at[pl.ds(_RPT * _NS, _RTAIL)],
                            outhi_hbm.at[pl.ds(_RPT * _NS, _RTAIL)])


_R = 512
_GRID = (_N + _R - 1) // _R


def _prep_body(deg0_ref, deg1_ref, x_ref, ulo_ref, uhi_ref, dinv_ref):
    deg = deg0_ref[:, 0:1] + deg1_ref[:, 0:1] + 1.0
    dinv = lax.rsqrt(deg)
    dinv_ref[...] = dinv
    ulo_ref[...] = x_ref[:, :_F] * dinv
    uhi_ref[...] = x_ref[:, _F:] * dinv


def _prep_call(deg0, deg1, x):
    return pl.pallas_call(
        _prep_body,
        grid=(_GRID,),
        in_specs=[
            pl.BlockSpec((_R, 16), lambda r: (r, 0)),
            pl.BlockSpec((_R, 16), lambda r: (r, 0)),
            pl.BlockSpec((_R, 2 * _F), lambda r: (r, 0)),
        ],
        out_specs=[
            pl.BlockSpec((_R, _F), lambda r: (r, 0)),
            pl.BlockSpec((_R, _F), lambda r: (r, 0)),
            pl.BlockSpec((_R, 1), lambda r: (r, 0)),
        ],
        out_shape=[
            jax.ShapeDtypeStruct((_N, _F), jnp.float32),
            jax.ShapeDtypeStruct((_N, _F), jnp.float32),
            jax.ShapeDtypeStruct((_N, 1), jnp.float32),
        ],
    )(deg0, deg1, x)


def _mid_body(slo_ref, shi_ref, ulo_ref, uhi_ref, dinv_ref, w1_ref, b1_ref,
              w2_ref, olo_ref, ohi_ref):
    dv = dinv_ref[...]
    alo = (slo_ref[...] + ulo_ref[...]) * dv
    ahi = (shi_ref[...] + uhi_ref[...]) * dv
    h = jnp.dot(alo, w1_ref[:_F, :], preferred_element_type=jnp.float32)
    h = h + jnp.dot(ahi, w1_ref[_F:, :], preferred_element_type=jnp.float32)
    h = jnp.maximum(h + b1_ref[...], 0.0)
    g = jnp.dot(h, w2_ref[...], preferred_element_type=jnp.float32)
    olo_ref[...] = g[:, :_F] * dv
    ohi_ref[...] = g[:, _F:] * dv


def _mid_call(slo, shi, ulo, uhi, dinv, W1, b1, W2):
    return pl.pallas_call(
        _mid_body,
        grid=(_GRID,),
        in_specs=[
            pl.BlockSpec((_R, _F), lambda r: (r, 0)),
            pl.BlockSpec((_R, _F), lambda r: (r, 0)),
            pl.BlockSpec((_R, _F), lambda r: (r, 0)),
            pl.BlockSpec((_R, _F), lambda r: (r, 0)),
            pl.BlockSpec((_R, 1), lambda r: (r, 0)),
            pl.BlockSpec((2 * _F, 512), lambda r: (0, 0)),
            pl.BlockSpec((1, 512), lambda r: (0, 0)),
            pl.BlockSpec((512, 2 * _F), lambda r: (0, 0)),
        ],
        out_specs=[
            pl.BlockSpec((_R, _F), lambda r: (r, 0)),
            pl.BlockSpec((_R, _F), lambda r: (r, 0)),
        ],
        out_shape=[
            jax.ShapeDtypeStruct((_N, _F), jnp.float32),
            jax.ShapeDtypeStruct((_N, _F), jnp.float32),
        ],
    )(slo, shi, ulo, uhi, dinv, W1, b1, W2)


def _final_body(slo_ref, shi_ref, ulo_ref, uhi_ref, dinv_ref, b2_ref,
                out_ref):
    dv = dinv_ref[...]
    out_ref[:, :_F] = jax.nn.sigmoid((slo_ref[...] + ulo_ref[...]) * dv
                                     + b2_ref[:, :_F])
    out_ref[:, _F:] = jax.nn.sigmoid((shi_ref[...] + uhi_ref[...]) * dv
                                     + b2_ref[:, _F:])


def _final_call(slo, shi, ulo, uhi, dinv, b2):
    return pl.pallas_call(
        _final_body,
        grid=(_GRID,),
        in_specs=[
            pl.BlockSpec((_R, _F), lambda r: (r, 0)),
            pl.BlockSpec((_R, _F), lambda r: (r, 0)),
            pl.BlockSpec((_R, _F), lambda r: (r, 0)),
            pl.BlockSpec((_R, _F), lambda r: (r, 0)),
            pl.BlockSpec((_R, 1), lambda r: (r, 0)),
            pl.BlockSpec((1, 2 * _F), lambda r: (0, 0)),
        ],
        out_specs=pl.BlockSpec((_R, 2 * _F), lambda r: (r, 0)),
        out_shape=jax.ShapeDtypeStruct((_N, 2 * _F), jnp.float32),
    )(slo, shi, ulo, uhi, dinv, b2)


def kernel(x, edge_index, W1, b1, W2, b2):
    src = edge_index[0].astype(jnp.int32)
    dst = edge_index[1].astype(jnp.int32)
    srcp = jnp.pad(src, (0, _EP - _E)).reshape(_EROWS, _CH)
    dstp = jnp.pad(dst, (0, _EP - _E),
                   constant_values=_N).reshape(_EROWS, _CH)

    deg0, deg1 = _deg_kernel(dstp)
    ulo, uhi, dinv = _prep_call(deg0, deg1, x)
    s1lo, s1hi = _scatter_kernel(ulo, uhi, srcp, dstp)
    u2lo, u2hi = _mid_call(s1lo, s1hi, ulo, uhi, dinv, W1,
                           b1.reshape(1, -1), W2)
    s2lo, s2hi = _scatter_kernel(u2lo, u2hi, srcp, dstp)
    return _final_call(s2lo, s2hi, u2lo, u2hi, dinv, b2.reshape(1, -1))


# speedup vs baseline: 1.0196x; 1.0196x over previous
import functools

import jax
import jax.numpy as jnp
from jax import lax
from jax.experimental import pallas as pl
from jax.experimental.pallas import tpu as pltpu
from jax.experimental.pallas import tpu_sc as plsc

_N = 10000
_E = 160000
_F = 128
_NC = 2
_NS = 16
_W = _NC * _NS

_CH = 128
_EP = 163840
_EROWS = _EP // _CH
_TROWS = _EROWS // _NS
_DROWS = _EROWS // _W
_RPT = 9984 // _NS
_RTAIL = _N - _RPT * _NS

_mesh = plsc.VectorSubcoreMesh(core_axis_name="c", subcore_axis_name="s")


@functools.partial(
    pl.kernel,
    mesh=_mesh,
    out_type=[
        jax.ShapeDtypeStruct((_N, 16), jnp.float32),
        jax.ShapeDtypeStruct((_N, 16), jnp.float32),
    ],
    scratch_types=[
        pltpu.VMEM((_DROWS, _CH), jnp.int32),
        pltpu.VMEM((_CH, 16), jnp.float32),
        pltpu.VMEM((_CH, 16), jnp.float32),
        pltpu.VMEM_SHARED((_N + 1, 16), jnp.float32),
        pltpu.SemaphoreType.DMA,
    ],
)
def _deg_kernel(dstp_hbm, out0_hbm, out1_hbm, dst_v, ones_v, zeros_v,
                acc_sh, sem):
    c = lax.axis_index("c")
    s = lax.axis_index("s")
    w = s * _NC + c

    zero16f = jnp.zeros((16,), jnp.float32)
    ones16f = jnp.ones((16,), jnp.float32)

    def fbody(i, carry):
        ones_v[i, pl.ds(0, 16)] = ones16f
        zeros_v[i, pl.ds(0, 16)] = zero16f
        return carry

    lax.fori_loop(0, _CH, fbody, 0)

    r0 = s * _RPT
    for k in range(_RPT // _CH):
        pltpu.sync_copy(zeros_v, acc_sh.at[pl.ds(r0 + k * _CH, _CH)])
    rem = _RPT % _CH
    pltpu.sync_copy(zeros_v.at[pl.ds(0, rem)],
                    acc_sh.at[pl.ds(r0 + (_RPT // _CH) * _CH, rem)])

    @pl.when(s == 0)
    def _():
        pltpu.sync_copy(zeros_v.at[pl.ds(0, _RTAIL)],
                        acc_sh.at[pl.ds(_RPT * _NS, _RTAIL)])

    pltpu.sync_copy(dstp_hbm.at[pl.ds(w * _DROWS, _DROWS)], dst_v)

    plsc.subcore_barrier()

    def body(j, carry):
        pltpu.async_copy(ones_v, acc_sh.at[dst_v.at[j]], sem, add=True)
        return carry

    lax.fori_loop(0, _DROWS, body, 0)

    def dbody(j, carry):
        pltpu.make_async_copy(ones_v, acc_sh.at[dst_v.at[j]], sem).wait()
        return carry

    lax.fori_loop(0, _DROWS, dbody, 0)

    plsc.subcore_barrier()

    @pl.when(c == 0)
    def _():
        pltpu.sync_copy(acc_sh.at[pl.ds(r0, _RPT)],
                        out0_hbm.at[pl.ds(r0, _RPT)])

        @pl.when(s == 0)
        def _():
            pltpu.sync_copy(acc_sh.at[pl.ds(_RPT * _NS, _RTAIL)],
                            out0_hbm.at[pl.ds(_RPT * _NS, _RTAIL)])

    @pl.when(c == 1)
    def _():
        pltpu.sync_copy(acc_sh.at[pl.ds(r0, _RPT)],
                        out1_hbm.at[pl.ds(r0, _RPT)])

        @pl.when(s == 0)
        def _():
            pltpu.sync_copy(acc_sh.at[pl.ds(_RPT * _NS, _RTAIL)],
                            out1_hbm.at[pl.ds(_RPT * _NS, _RTAIL)])


@functools.partial(
    pl.kernel,
    mesh=_mesh,
    out_type=[
        jax.ShapeDtypeStruct((_N, _F), jnp.float32),
        jax.ShapeDtypeStruct((_N, _F), jnp.float32),
    ],
    scratch_types=[
        pltpu.VMEM((_CH,), jnp.int32),
        pltpu.VMEM((_CH,), jnp.int32),
        pltpu.VMEM((_CH,), jnp.int32),
        pltpu.VMEM((_CH,), jnp.int32),
        pltpu.VMEM((_CH,), jnp.int32),
        pltpu.VMEM((_CH,), jnp.int32),
        pltpu.VMEM((_CH,), jnp.int32),
        pltpu.VMEM((_CH,), jnp.int32),
        pltpu.VMEM((_CH, _F), jnp.float32),
        pltpu.VMEM((_CH, _F), jnp.float32),
        pltpu.VMEM_SHARED((_N + 1, _F), jnp.float32),
        pltpu.SemaphoreType.DMA,
        pltpu.SemaphoreType.DMA,
        pltpu.SemaphoreType.DMA,
        pltpu.SemaphoreType.DMA,
        pltpu.SemaphoreType.DMA,
        pltpu.SemaphoreType.DMA,
    ],
)
def _scatter_kernel(ulo_hbm, uhi_hbm, srcp_hbm, dstp_hbm, outlo_hbm,
                    outhi_hbm, src_i0, dst_i0, src_i1, dst_i1, src_i2,
                    dst_i2, src_i3, dst_i3, rows_v0, rows_v1, acc_sh,
                    semg0, semg1, semi0, semi1, semi2, semi3):
    c = lax.axis_index("c")
    s = lax.axis_index("s")

    src_i = (src_i0, src_i1, src_i2, src_i3)
    dst_i = (dst_i0, dst_i1, dst_i2, dst_i3)
    semi = (semi0, semi1, semi2, semi3)
    rows = (rows_v0, rows_v1)
    semg = (semg0, semg1)
    rb = s * _TROWS

    def _fire_idx(m, b):
        off = (rb + m) * _CH
        pltpu.async_copy(srcp_hbm.at[pl.ds(off, _CH)], src_i[b], semi[b])
        pltpu.async_copy(dstp_hbm.at[pl.ds(off, _CH)], dst_i[b], semi[b])

    def _drain_idx(m, b):
        off = (rb + m) * _CH
        pltpu.make_async_copy(srcp_hbm.at[pl.ds(off, _CH)], src_i[b],
                              semi[b]).wait()
        pltpu.make_async_copy(dstp_hbm.at[pl.ds(off, _CH)], dst_i[b],
                              semi[b]).wait()

    def _fire(b, r):
        @pl.when(c == 0)
        def _():
            pltpu.async_copy(ulo_hbm.at[src_i[b]], rows[r], semg[r])

        @pl.when(c == 1)
        def _():
            pltpu.async_copy(uhi_hbm.at[src_i[b]], rows[r], semg[r])

    def _drain(b, r):
        @pl.when(c == 0)
        def _():
            pltpu.make_async_copy(ulo_hbm.at[src_i[b]], rows[r],
                                  semg[r]).wait()

        @pl.when(c == 1)
        def _():
            pltpu.make_async_copy(uhi_hbm.at[src_i[b]], rows[r],
                                  semg[r]).wait()

    zero16f = jnp.zeros((16,), jnp.float32)

    def zbody(i, carry):
        for j in range(_F // 16):
            rows_v0[i, pl.ds(j * 16, 16)] = zero16f
        return carry

    lax.fori_loop(0, _CH, zbody, 0)

    r0 = s * _RPT
    for k in range(_RPT // _CH):
        pltpu.sync_copy(rows_v0, acc_sh.at[pl.ds(r0 + k * _CH, _CH)])
    rem = _RPT % _CH
    pltpu.sync_copy(rows_v0.at[pl.ds(0, rem)],
                    acc_sh.at[pl.ds(r0 + (_RPT // _CH) * _CH, rem)])

    @pl.when(s == 0)
    def _():
        pltpu.sync_copy(rows_v0.at[pl.ds(0, _RTAIL)],
                        acc_sh.at[pl.ds(_RPT * _NS, _RTAIL)])

    plsc.subcore_barrier()

    _fire_idx(0, 0)
    _fire_idx(1, 1)
    _drain_idx(0, 0)
    _fire(0, 0)

    def body(k, carry):
        for u in range(4):
            j = 4 * k + u
            r = u % 2
            b = u
            bn = (u + 1) % 4
            bp = (u + 2) % 4

            @pl.when(j + 2 < _TROWS)
            def _():
                _fire_idx(j + 2, bp)

            @pl.when(j + 1 < _TROWS)
            def _():
                _drain_idx(j + 1, bn)
                _fire(bn, 1 - r)

            _drain(b, r)
            pltpu.sync_copy(rows[r], acc_sh.at[dst_i[b]], add=True)
        return carry

    lax.fori_loop(0, _TROWS // 4, body, 0)

    plsc.subcore_barrier()

    @pl.when(c == 0)
    def _():
        pltpu.sync_copy(acc_sh.at[pl.ds(r0, _RPT)],
                        outlo_hbm.at[pl.ds(r0, _RPT)])

        @pl.when(s == 0)
        def _():
            pltpu.sync_copy(acc_sh.at[pl.ds(_RPT * _NS, _RTAIL)],
                            outlo_hbm.at[pl.ds(_RPT * _NS, _RTAIL)])

    @pl.when(c == 1)
    def _():
        pltpu.sync_copy(acc_sh.at[pl.ds(r0, _RPT)],
                        outhi_hbm.at[pl.ds(r0, _RPT)])

        @pl.when(s == 0)
        def _():
            pltpu.sync_copy(acc_sh.at[pl.ds(_RPT * _NS, _RTAIL)],
                            outhi_hbm.at[pl.ds(_RPT * _NS, _RTAIL)])


_R = 512
_GRID = (_N + _R - 1) // _R


def _prep_body(deg0_ref, deg1_ref, x_ref, ulo_ref, uhi_ref, dinv_ref):
    deg = deg0_ref[:, 0:1] + deg1_ref[:, 0:1] + 1.0
    dinv = lax.rsqrt(deg)
    dinv_ref[...] = dinv
    ulo_ref[...] = x_ref[:, :_F] * dinv
    uhi_ref[...] = x_ref[:, _F:] * dinv


def _prep_call(deg0, deg1, x):
    return pl.pallas_call(
        _prep_body,
        grid=(_GRID,),
        in_specs=[
            pl.BlockSpec((_R, 16), lambda r: (r, 0)),
            pl.BlockSpec((_R, 16), lambda r: (r, 0)),
            pl.BlockSpec((_R, 2 * _F), lambda r: (r, 0)),
        ],
        out_specs=[
            pl.BlockSpec((_R, _F), lambda r: (r, 0)),
            pl.BlockSpec((_R, _F), lambda r: (r, 0)),
            pl.BlockSpec((_R, 1), lambda r: (r, 0)),
        ],
        out_shape=[
            jax.ShapeDtypeStruct((_N, _F), jnp.float32),
            jax.ShapeDtypeStruct((_N, _F), jnp.float32),
            jax.ShapeDtypeStruct((_N, 1), jnp.float32),
        ],
    )(deg0, deg1, x)


def _mid_body(slo_ref, shi_ref, ulo_ref, uhi_ref, dinv_ref, w1_ref, b1_ref,
              w2_ref, olo_ref, ohi_ref):
    dv = dinv_ref[...]
    alo = (slo_ref[...] + ulo_ref[...]) * dv
    ahi = (shi_ref[...] + uhi_ref[...]) * dv
    h = jnp.dot(alo, w1_ref[:_F, :], preferred_element_type=jnp.float32)
    h = h + jnp.dot(ahi, w1_ref[_F:, :], preferred_element_type=jnp.float32)
    h = jnp.maximum(h + b1_ref[...], 0.0)
    g = jnp.dot(h, w2_ref[...], preferred_element_type=jnp.float32)
    olo_ref[...] = g[:, :_F] * dv
    ohi_ref[...] = g[:, _F:] * dv


def _mid_call(slo, shi, ulo, uhi, dinv, W1, b1, W2):
    return pl.pallas_call(
        _mid_body,
        grid=(_GRID,),
        in_specs=[
            pl.BlockSpec((_R, _F), lambda r: (r, 0)),
            pl.BlockSpec((_R, _F), lambda r: (r, 0)),
            pl.BlockSpec((_R, _F), lambda r: (r, 0)),
            pl.BlockSpec((_R, _F), lambda r: (r, 0)),
            pl.BlockSpec((_R, 1), lambda r: (r, 0)),
            pl.BlockSpec((2 * _F, 512), lambda r: (0, 0)),
            pl.BlockSpec((1, 512), lambda r: (0, 0)),
            pl.BlockSpec((512, 2 * _F), lambda r: (0, 0)),
        ],
        out_specs=[
            pl.BlockSpec((_R, _F), lambda r: (r, 0)),
            pl.BlockSpec((_R, _F), lambda r: (r, 0)),
        ],
        out_shape=[
            jax.ShapeDtypeStruct((_N, _F), jnp.float32),
            jax.ShapeDtypeStruct((_N, _F), jnp.float32),
        ],
    )(slo, shi, ulo, uhi, dinv, W1, b1, W2)


def _final_body(slo_ref, shi_ref, ulo_ref, uhi_ref, dinv_ref, b2_ref,
                out_ref):
    dv = dinv_ref[...]
    out_ref[:, :_F] = jax.nn.sigmoid((slo_ref[...] + ulo_ref[...]) * dv
                                     + b2_ref[:, :_F])
    out_ref[:, _F:] = jax.nn.sigmoid((shi_ref[...] + uhi_ref[...]) * dv
                                     + b2_ref[:, _F:])


def _final_call(slo, shi, ulo, uhi, dinv, b2):
    return pl.pallas_call(
        _final_body,
        grid=(_GRID,),
        in_specs=[
            pl.BlockSpec((_R, _F), lambda r: (r, 0)),
            pl.BlockSpec((_R, _F), lambda r: (r, 0)),
            pl.BlockSpec((_R, _F), lambda r: (r, 0)),
            pl.BlockSpec((_R, _F), lambda r: (r, 0)),
            pl.BlockSpec((_R, 1), lambda r: (r, 0)),
            pl.BlockSpec((1, 2 * _F), lambda r: (0, 0)),
        ],
        out_specs=pl.BlockSpec((_R, 2 * _F), lambda r: (r, 0)),
        out_shape=jax.ShapeDtypeStruct((_N, 2 * _F), jnp.float32),
    )(slo, shi, ulo, uhi, dinv, b2)


def kernel(x, edge_index, W1, b1, W2, b2):
    src = edge_index[0].astype(jnp.int32)
    dst = edge_index[1].astype(jnp.int32)
    srcp = jnp.pad(src, (0, _EP - _E)).reshape(_EROWS, _CH)
    dstp = jnp.pad(dst, (0, _EP - _E),
                   constant_values=_N).reshape(_EROWS, _CH)

    srcp1 = srcp.reshape(_EP)
    dstp1 = dstp.reshape(_EP)

    deg0, deg1 = _deg_kernel(dstp)
    ulo, uhi, dinv = _prep_call(deg0, deg1, x)
    s1lo, s1hi = _scatter_kernel(ulo, uhi, srcp1, dstp1)
    u2lo, u2hi = _mid_call(s1lo, s1hi, ulo, uhi, dinv, W1,
                           b1.reshape(1, -1), W2)
    s2lo, s2hi = _scatter_kernel(u2lo, u2hi, srcp1, dstp1)
    return _final_call(s2lo, s2hi, u2lo, u2hi, dinv, b2.reshape(1, -1))


# speedup vs baseline: 2.1840x; 2.1420x over previous
import functools

import jax
import jax.numpy as jnp
from jax import lax
from jax.experimental import pallas as pl
from jax.experimental.pallas import tpu as pltpu
from jax.experimental.pallas import tpu_sc as plsc

_N = 10000
_E = 160000
_F = 128
_NC = 2
_NS = 16
_W = _NC * _NS

_CH = 128
_EP = 163840
_EROWS = _EP // _CH
_TROWS = _EROWS // _NS
_DROWS = _EROWS // _W
_RPT = 9984 // _NS
_RTAIL = _N - _RPT * _NS

_mesh = plsc.VectorSubcoreMesh(core_axis_name="c", subcore_axis_name="s")


@functools.partial(
    pl.kernel,
    mesh=_mesh,
    out_type=[
        jax.ShapeDtypeStruct((_N, 16), jnp.float32),
        jax.ShapeDtypeStruct((_N, 16), jnp.float32),
    ],
    scratch_types=[
        pltpu.VMEM((_DROWS, _CH), jnp.int32),
        pltpu.VMEM((_CH, 16), jnp.float32),
        pltpu.VMEM((_CH, 16), jnp.float32),
        pltpu.VMEM_SHARED((_N + 128, 16), jnp.float32),
        pltpu.SemaphoreType.DMA,
    ],
)
def _deg_kernel(dstp_hbm, out0_hbm, out1_hbm, dst_v, ones_v, zeros_v,
                acc_sh, sem):
    c = lax.axis_index("c")
    s = lax.axis_index("s")
    w = s * _NC + c

    zero16f = jnp.zeros((16,), jnp.float32)
    ones16f = jnp.ones((16,), jnp.float32)

    def fbody(i, carry):
        ones_v[i, pl.ds(0, 16)] = ones16f
        zeros_v[i, pl.ds(0, 16)] = zero16f
        return carry

    lax.fori_loop(0, _CH, fbody, 0)

    r0 = s * _RPT
    for k in range(_RPT // _CH):
        pltpu.sync_copy(zeros_v, acc_sh.at[pl.ds(r0 + k * _CH, _CH)])
    rem = _RPT % _CH
    pltpu.sync_copy(zeros_v.at[pl.ds(0, rem)],
                    acc_sh.at[pl.ds(r0 + (_RPT // _CH) * _CH, rem)])

    @pl.when(s == 0)
    def _():
        pltpu.sync_copy(zeros_v.at[pl.ds(0, _RTAIL)],
                        acc_sh.at[pl.ds(_RPT * _NS, _RTAIL)])

    pltpu.sync_copy(dstp_hbm.at[pl.ds(w * _DROWS, _DROWS)], dst_v)

    plsc.subcore_barrier()

    def body(j, carry):
        pltpu.async_copy(ones_v, acc_sh.at[dst_v.at[j]], sem, add=True)
        return carry

    lax.fori_loop(0, _DROWS, body, 0)

    def dbody(j, carry):
        pltpu.make_async_copy(ones_v, acc_sh.at[dst_v.at[j]], sem).wait()
        return carry

    lax.fori_loop(0, _DROWS, dbody, 0)

    plsc.subcore_barrier()

    @pl.when(c == 0)
    def _():
        pltpu.sync_copy(acc_sh.at[pl.ds(r0, _RPT)],
                        out0_hbm.at[pl.ds(r0, _RPT)])

        @pl.when(s == 0)
        def _():
            pltpu.sync_copy(acc_sh.at[pl.ds(_RPT * _NS, _RTAIL)],
                            out0_hbm.at[pl.ds(_RPT * _NS, _RTAIL)])

    @pl.when(c == 1)
    def _():
        pltpu.sync_copy(acc_sh.at[pl.ds(r0, _RPT)],
                        out1_hbm.at[pl.ds(r0, _RPT)])

        @pl.when(s == 0)
        def _():
            pltpu.sync_copy(acc_sh.at[pl.ds(_RPT * _NS, _RTAIL)],
                            out1_hbm.at[pl.ds(_RPT * _NS, _RTAIL)])


@functools.partial(
    pl.kernel,
    mesh=_mesh,
    out_type=[
        jax.ShapeDtypeStruct((_N, _F), jnp.float32),
        jax.ShapeDtypeStruct((_N, _F), jnp.float32),
    ],
    scratch_types=[
        pltpu.VMEM((_CH,), jnp.int32),
        pltpu.VMEM((_CH,), jnp.int32),
        pltpu.VMEM((_CH,), jnp.int32),
        pltpu.VMEM((_CH,), jnp.int32),
        pltpu.VMEM((_CH,), jnp.int32),
        pltpu.VMEM((_CH,), jnp.int32),
        pltpu.VMEM((_CH,), jnp.int32),
        pltpu.VMEM((_CH,), jnp.int32),
        pltpu.VMEM((_CH, _F), jnp.float32),
        pltpu.VMEM((_CH, _F), jnp.float32),
        pltpu.VMEM_SHARED((_N + 128, _F), jnp.float32),
        pltpu.SemaphoreType.DMA,
        pltpu.SemaphoreType.DMA,
        pltpu.SemaphoreType.DMA,
        pltpu.SemaphoreType.DMA,
        pltpu.SemaphoreType.DMA,
        pltpu.SemaphoreType.DMA,
    ],
)
def _scatter_kernel(ulo_hbm, uhi_hbm, srcp_hbm, dstp_hbm, outlo_hbm,
                    outhi_hbm, src_i0, dst_i0, src_i1, dst_i1, src_i2,
                    dst_i2, src_i3, dst_i3, rows_v0, rows_v1, acc_sh,
                    semg0, semg1, semi0, semi1, semi2, semi3):
    c = lax.axis_index("c")
    s = lax.axis_index("s")

    src_i = (src_i0, src_i1, src_i2, src_i3)
    dst_i = (dst_i0, dst_i1, dst_i2, dst_i3)
    semi = (semi0, semi1, semi2, semi3)
    rows = (rows_v0, rows_v1)
    semg = (semg0, semg1)
    rb = s * _TROWS

    def _fire_idx(m, b):
        off = (rb + m) * _CH
        pltpu.async_copy(srcp_hbm.at[pl.ds(off, _CH)], src_i[b], semi[b])
        pltpu.async_copy(dstp_hbm.at[pl.ds(off, _CH)], dst_i[b], semi[b])

    def _drain_idx(m, b):
        off = (rb + m) * _CH
        pltpu.make_async_copy(srcp_hbm.at[pl.ds(off, _CH)], src_i[b],
                              semi[b]).wait()
        pltpu.make_async_copy(dstp_hbm.at[pl.ds(off, _CH)], dst_i[b],
                              semi[b]).wait()

    def _fire(b, r):
        @pl.when(c == 0)
        def _():
            pltpu.async_copy(ulo_hbm.at[src_i[b]], rows[r], semg[r])

        @pl.when(c == 1)
        def _():
            pltpu.async_copy(uhi_hbm.at[src_i[b]], rows[r], semg[r])

    def _drain(b, r):
        @pl.when(c == 0)
        def _():
            pltpu.make_async_copy(ulo_hbm.at[src_i[b]], rows[r],
                                  semg[r]).wait()

        @pl.when(c == 1)
        def _():
            pltpu.make_async_copy(uhi_hbm.at[src_i[b]], rows[r],
                                  semg[r]).wait()

    zero16f = jnp.zeros((16,), jnp.float32)

    def zbody(i, carry):
        for j in range(_F // 16):
            rows_v0[i, pl.ds(j * 16, 16)] = zero16f
        return carry

    lax.fori_loop(0, _CH, zbody, 0)

    r0 = s * _RPT
    for k in range(_RPT // _CH):
        pltpu.sync_copy(rows_v0, acc_sh.at[pl.ds(r0 + k * _CH, _CH)])
    rem = _RPT % _CH
    pltpu.sync_copy(rows_v0.at[pl.ds(0, rem)],
                    acc_sh.at[pl.ds(r0 + (_RPT // _CH) * _CH, rem)])

    @pl.when(s == 0)
    def _():
        pltpu.sync_copy(rows_v0.at[pl.ds(0, _RTAIL)],
                        acc_sh.at[pl.ds(_RPT * _NS, _RTAIL)])

    plsc.subcore_barrier()

    _fire_idx(0, 0)
    _fire_idx(1, 1)
    _drain_idx(0, 0)
    _fire(0, 0)

    def body(k, carry):
        for u in range(4):
            j = 4 * k + u
            r = u % 2
            b = u
            bn = (u + 1) % 4
            bp = (u + 2) % 4

            @pl.when(j + 2 < _TROWS)
            def _():
                _fire_idx(j + 2, bp)

            @pl.when(j + 1 < _TROWS)
            def _():
                _drain_idx(j + 1, bn)
                _fire(bn, 1 - r)

            _drain(b, r)
            pltpu.sync_copy(rows[r], acc_sh.at[dst_i[b]], add=True)
        return carry

    lax.fori_loop(0, _TROWS // 4, body, 0)

    plsc.subcore_barrier()

    @pl.when(c == 0)
    def _():
        pltpu.sync_copy(acc_sh.at[pl.ds(r0, _RPT)],
                        outlo_hbm.at[pl.ds(r0, _RPT)])

        @pl.when(s == 0)
        def _():
            pltpu.sync_copy(acc_sh.at[pl.ds(_RPT * _NS, _RTAIL)],
                            outlo_hbm.at[pl.ds(_RPT * _NS, _RTAIL)])

    @pl.when(c == 1)
    def _():
        pltpu.sync_copy(acc_sh.at[pl.ds(r0, _RPT)],
                        outhi_hbm.at[pl.ds(r0, _RPT)])

        @pl.when(s == 0)
        def _():
            pltpu.sync_copy(acc_sh.at[pl.ds(_RPT * _NS, _RTAIL)],
                            outhi_hbm.at[pl.ds(_RPT * _NS, _RTAIL)])


_R = 512
_GRID = (_N + _R - 1) // _R


def _prep_body(deg0_ref, deg1_ref, x_ref, ulo_ref, uhi_ref, dinv_ref):
    deg = deg0_ref[:, 0:1] + deg1_ref[:, 0:1] + 1.0
    dinv = lax.rsqrt(deg)
    dinv_ref[...] = dinv
    ulo_ref[...] = x_ref[:, :_F] * dinv
    uhi_ref[...] = x_ref[:, _F:] * dinv


def _prep_call(deg0, deg1, x):
    return pl.pallas_call(
        _prep_body,
        grid=(_GRID,),
        in_specs=[
            pl.BlockSpec((_R, 16), lambda r: (r, 0)),
            pl.BlockSpec((_R, 16), lambda r: (r, 0)),
            pl.BlockSpec((_R, 2 * _F), lambda r: (r, 0)),
        ],
        out_specs=[
            pl.BlockSpec((_R, _F), lambda r: (r, 0)),
            pl.BlockSpec((_R, _F), lambda r: (r, 0)),
            pl.BlockSpec((_R, 1), lambda r: (r, 0)),
        ],
        out_shape=[
            jax.ShapeDtypeStruct((_N, _F), jnp.float32),
            jax.ShapeDtypeStruct((_N, _F), jnp.float32),
            jax.ShapeDtypeStruct((_N, 1), jnp.float32),
        ],
    )(deg0, deg1, x)


def _mid_body(slo_ref, shi_ref, ulo_ref, uhi_ref, dinv_ref, w1_ref, b1_ref,
              w2_ref, olo_ref, ohi_ref):
    dv = dinv_ref[...]
    alo = (slo_ref[...] + ulo_ref[...]) * dv
    ahi = (shi_ref[...] + uhi_ref[...]) * dv
    h = jnp.dot(alo, w1_ref[:_F, :], preferred_element_type=jnp.float32)
    h = h + jnp.dot(ahi, w1_ref[_F:, :], preferred_element_type=jnp.float32)
    h = jnp.maximum(h + b1_ref[...], 0.0)
    g = jnp.dot(h, w2_ref[...], preferred_element_type=jnp.float32)
    olo_ref[...] = g[:, :_F] * dv
    ohi_ref[...] = g[:, _F:] * dv


def _mid_call(slo, shi, ulo, uhi, dinv, W1, b1, W2):
    return pl.pallas_call(
        _mid_body,
        grid=(_GRID,),
        in_specs=[
            pl.BlockSpec((_R, _F), lambda r: (r, 0)),
            pl.BlockSpec((_R, _F), lambda r: (r, 0)),
            pl.BlockSpec((_R, _F), lambda r: (r, 0)),
            pl.BlockSpec((_R, _F), lambda r: (r, 0)),
            pl.BlockSpec((_R, 1), lambda r: (r, 0)),
            pl.BlockSpec((2 * _F, 512), lambda r: (0, 0)),
            pl.BlockSpec((1, 512), lambda r: (0, 0)),
            pl.BlockSpec((512, 2 * _F), lambda r: (0, 0)),
        ],
        out_specs=[
            pl.BlockSpec((_R, _F), lambda r: (r, 0)),
            pl.BlockSpec((_R, _F), lambda r: (r, 0)),
        ],
        out_shape=[
            jax.ShapeDtypeStruct((_N, _F), jnp.float32),
            jax.ShapeDtypeStruct((_N, _F), jnp.float32),
        ],
    )(slo, shi, ulo, uhi, dinv, W1, b1, W2)


def _final_body(slo_ref, shi_ref, ulo_ref, uhi_ref, dinv_ref, b2_ref,
                out_ref):
    dv = dinv_ref[...]
    out_ref[:, :_F] = jax.nn.sigmoid((slo_ref[...] + ulo_ref[...]) * dv
                                     + b2_ref[:, :_F])
    out_ref[:, _F:] = jax.nn.sigmoid((shi_ref[...] + uhi_ref[...]) * dv
                                     + b2_ref[:, _F:])


def _final_call(slo, shi, ulo, uhi, dinv, b2):
    return pl.pallas_call(
        _final_body,
        grid=(_GRID,),
        in_specs=[
            pl.BlockSpec((_R, _F), lambda r: (r, 0)),
            pl.BlockSpec((_R, _F), lambda r: (r, 0)),
            pl.BlockSpec((_R, _F), lambda r: (r, 0)),
            pl.BlockSpec((_R, _F), lambda r: (r, 0)),
            pl.BlockSpec((_R, 1), lambda r: (r, 0)),
            pl.BlockSpec((1, 2 * _F), lambda r: (0, 0)),
        ],
        out_specs=pl.BlockSpec((_R, 2 * _F), lambda r: (r, 0)),
        out_shape=jax.ShapeDtypeStruct((_N, 2 * _F), jnp.float32),
    )(slo, shi, ulo, uhi, dinv, b2)


def kernel(x, edge_index, W1, b1, W2, b2):
    src = edge_index[0].astype(jnp.int32)
    dst = edge_index[1].astype(jnp.int32)
    pad_idx = jnp.arange(_EP - _E, dtype=jnp.int32) % 128
    srcp = jnp.concatenate([src, pad_idx]).reshape(_EROWS, _CH)
    dstp = jnp.concatenate([dst, _N + pad_idx]).reshape(_EROWS, _CH)

    srcp1 = srcp.reshape(_EP)
    dstp1 = dstp.reshape(_EP)

    deg0, deg1 = _deg_kernel(dstp)
    ulo, uhi, dinv = _prep_call(deg0, deg1, x)
    s1lo, s1hi = _scatter_kernel(ulo, uhi, srcp1, dstp1)
    u2lo, u2hi = _mid_call(s1lo, s1hi, ulo, uhi, dinv, W1,
                           b1.reshape(1, -1), W2)
    s2lo, s2hi = _scatter_kernel(u2lo, u2hi, srcp1, dstp1)
    return _final_call(s2lo, s2hi, u2lo, u2hi, dinv, b2.reshape(1, -1))
